# Initial kernel scaffold; baseline (speedup 1.0000x reference)
import jax
import jax.numpy as jnp
from jax.experimental import pallas as pl


def kernel(src_cams, src_verts, faces, src_fim, tgt_fim):
    raise NotImplementedError("write your pallas kernel here")


# speedup vs baseline: 42.4467x; 42.4467x over previous
import functools

import jax
import jax.numpy as jnp
from jax import lax
from jax.experimental import pallas as pl
from jax.experimental.pallas import tpu as pltpu
from jax.experimental.pallas import tpu_sc as plsc

BS = 8
H = 256
NV = 6890
NF = 13776
P = H * H

NC = 2
NS = 16
L = 16
W = NC * NS
WPB = W // BS
PPW = P // WPB
CH = 2048

VPAD = 20688
PXPAD = 6896


def _body(cams_hbm, verts_hbm, faces_hbm, fim_hbm, out_hbm,
          cams_v, verts_v, faces_v, px_v, py_v, bcx_v, bcy_v, idx_v, out_v):
    s = lax.axis_index("s")
    c = lax.axis_index("c")
    wid = s * NC + c
    b = wid // WPB
    q = wid % WPB

    pltpu.sync_copy(cams_hbm, cams_v)
    pltpu.sync_copy(verts_hbm.at[b], verts_v)
    pltpu.sync_copy(faces_hbm, faces_v)

    iota = lax.iota(jnp.int32, L)
    iota3 = iota * 3

    csplat = iota * 0 + 3 * b
    cam0 = plsc.load_gather(cams_v, [csplat])
    cam1 = plsc.load_gather(cams_v, [csplat + 1])
    cam2 = plsc.load_gather(cams_v, [csplat + 2])

    def p1(i, carry):
        xi = iota3 + i * (L * 3)
        xv = plsc.load_gather(verts_v, [xi])
        yv = plsc.load_gather(verts_v, [xi + 1])
        px_v[pl.ds(i * L, L)] = cam0 * (xv + cam1)
        py_v[pl.ds(i * L, L)] = cam0 * (yv + cam2)
        return carry

    lax.fori_loop(0, PXPAD // L, p1, 0, unroll=4)

    def p2(i, carry):
        fi = iota3 + i * (L * 3)
        f0 = plsc.load_gather(faces_v, [fi])
        f1 = plsc.load_gather(faces_v, [fi + 1])
        f2 = plsc.load_gather(faces_v, [fi + 2])
        sx = (plsc.load_gather(px_v, [f0]) + plsc.load_gather(px_v, [f1])
              + plsc.load_gather(px_v, [f2]))
        sy = (plsc.load_gather(py_v, [f0]) + plsc.load_gather(py_v, [f1])
              + plsc.load_gather(py_v, [f2]))
        bcx_v[pl.ds(i * L, L)] = sx * jnp.float32(1.0 / 3.0)
        bcy_v[pl.ds(i * L, L)] = sy * jnp.float32(1.0 / 3.0)
        return carry

    lax.fori_loop(0, NF // L, p2, 0, unroll=4)

    pix0 = q * PPW

    def chunk(ci, carry):
        start = pix0 + ci * CH
        pltpu.sync_copy(fim_hbm.at[b, pl.ds(start, CH)], idx_v)

        def inner(j, icarry):
            off = j * L
            t = idx_v[pl.ds(off, L)]
            mask = t >= 0
            tc = jnp.minimum(jnp.maximum(t, 0), NF - 1)
            gx = plsc.load_gather(bcx_v, [tc])
            gy = plsc.load_gather(bcy_v, [tc])
            rx = jnp.where(mask, gx, jnp.float32(-1.0))
            ry = jnp.where(mask, gy, jnp.float32(-1.0))
            sidx = iota * 2 + off * 2
            plsc.store_scatter(out_v, [sidx], rx)
            plsc.store_scatter(out_v, [sidx + 1], ry)
            return icarry

        lax.fori_loop(0, CH // L, inner, 0, unroll=4)
        pltpu.sync_copy(out_v, out_hbm.at[b, pl.ds(start * 2, CH * 2)])
        return carry

    lax.fori_loop(0, PPW // CH, chunk, 0)


@functools.partial(jax.jit, static_argnames=())
def _run(cams, verts, faces_f, fim):
    mesh = plsc.VectorSubcoreMesh(core_axis_name="c", subcore_axis_name="s",
                                  num_cores=NC, num_subcores=NS)
    f = pl.kernel(
        _body,
        out_type=jax.ShapeDtypeStruct((BS, 2 * P), jnp.float32),
        mesh=mesh,
        compiler_params=pltpu.CompilerParams(needs_layout_passes=False),
        scratch_types=[
            pltpu.VMEM((32,), jnp.float32),
            pltpu.VMEM((VPAD,), jnp.float32),
            pltpu.VMEM((NF * 3,), jnp.int32),
            pltpu.VMEM((PXPAD,), jnp.float32),
            pltpu.VMEM((PXPAD,), jnp.float32),
            pltpu.VMEM((NF,), jnp.float32),
            pltpu.VMEM((NF,), jnp.float32),
            pltpu.VMEM((CH,), jnp.int32),
            pltpu.VMEM((2 * CH,), jnp.float32),
        ],
    )
    return f(cams, verts, faces_f, fim)


def kernel(src_cams, src_verts, faces, src_fim, tgt_fim):
    del src_fim
    cams = jnp.pad(src_cams.astype(jnp.float32).reshape(-1), (0, 32 - 3 * BS))
    verts = jnp.pad(src_verts.astype(jnp.float32).reshape(BS, NV * 3),
                    ((0, 0), (0, VPAD - NV * 3)))
    faces_f = faces.astype(jnp.int32).reshape(-1)
    fim = tgt_fim.astype(jnp.int32).reshape(BS, P)
    out = _run(cams, verts, faces_f, fim)
    return out.reshape(BS, H, H, 2)


# speedup vs baseline: 77.5159x; 1.8262x over previous
import jax
import jax.numpy as jnp
from jax import lax
from jax.experimental import pallas as pl
from jax.experimental.pallas import tpu as pltpu
from jax.experimental.pallas import tpu_sc as plsc

BS = 8
H = 256
NV = 6890
NF = 13776
P = H * H

NC = 2
NS = 16
L = 16
W = NC * NS
WPB = W // BS
RPW = H // WPB
RCH = 8
NVUP = 6896
VPAD = 20688


def _body(cams_hbm, verts_hbm, faces_hbm, fim_hbm, out_hbm,
          cams_v, verts_v, faces_v, px_v, py_v, bcx_v, bcy_v, fim_v, out_v):
    s = lax.axis_index("s")
    c = lax.axis_index("c")
    wid = s * NC + c
    b = wid // WPB
    q = wid % WPB

    pltpu.sync_copy(cams_hbm, cams_v)
    pltpu.sync_copy(verts_hbm.at[b], verts_v)
    pltpu.sync_copy(faces_hbm, faces_v)

    iota = lax.iota(jnp.int32, L)
    zero = iota * 0

    bsplat = zero + b
    cam0 = plsc.load_gather(cams_v, [bsplat, zero])
    cam1 = plsc.load_gather(cams_v, [bsplat, zero + 1])
    cam2 = plsc.load_gather(cams_v, [bsplat, zero + 2])

    iota3 = iota * 3

    def p1(i, carry):
        xi = iota3 + i * (L * 3)
        xv = plsc.load_gather(verts_v, [xi])
        yv = plsc.load_gather(verts_v, [xi + 1])
        px_v[pl.ds(i * L, L)] = cam0 * (xv + cam1)
        py_v[pl.ds(i * L, L)] = cam0 * (yv + cam2)
        return carry

    lax.fori_loop(0, NVUP // L, p1, 0, unroll=4)

    def p2(i, carry):
        fi = iota3 + i * (L * 3)
        f0 = plsc.load_gather(faces_v, [fi])
        f1 = plsc.load_gather(faces_v, [fi + 1])
        f2 = plsc.load_gather(faces_v, [fi + 2])
        sx = (plsc.load_gather(px_v, [f0]) + plsc.load_gather(px_v, [f1])
              + plsc.load_gather(px_v, [f2]))
        sy = (plsc.load_gather(py_v, [f0]) + plsc.load_gather(py_v, [f1])
              + plsc.load_gather(py_v, [f2]))
        bcx_v[pl.ds(i * L, L)] = sx * jnp.float32(1.0 / 3.0)
        bcy_v[pl.ds(i * L, L)] = sy * jnp.float32(1.0 / 3.0)
        return carry

    lax.fori_loop(0, NF // L, p2, 0, unroll=4)

    row0 = q * RPW

    def chunk(ci, carry):
        r0 = row0 + ci * RCH
        pltpu.sync_copy(fim_hbm.at[b, pl.ds(r0, RCH)], fim_v)

        def row(r, rcarry):
            rsplat = zero + r

            def col(v, ccarry):
                t = fim_v[r, pl.ds(v * L, L)]
                mask = t >= 0
                tc = jnp.minimum(jnp.maximum(t, 0), NF - 1)
                gx = plsc.load_gather(bcx_v, [tc])
                gy = plsc.load_gather(bcy_v, [tc])
                rx = jnp.where(mask, gx, jnp.float32(-1.0))
                ry = jnp.where(mask, gy, jnp.float32(-1.0))
                cidx = iota * 2 + v * (L * 2)
                plsc.store_scatter(out_v, [rsplat, cidx], rx)
                plsc.store_scatter(out_v, [rsplat, cidx + 1], ry)
                return ccarry

            return lax.fori_loop(0, H // L, col, rcarry, unroll=4)

        lax.fori_loop(0, RCH, row, 0)
        pltpu.sync_copy(out_v, out_hbm.at[b, pl.ds(r0, RCH)])
        return carry

    lax.fori_loop(0, RPW // RCH, chunk, 0)


@jax.jit
def _run(cams, verts, faces_a, fim):
    mesh = plsc.VectorSubcoreMesh(core_axis_name="c", subcore_axis_name="s",
                                  num_cores=NC, num_subcores=NS)
    f = pl.kernel(
        _body,
        out_type=jax.ShapeDtypeStruct((BS, H, 2 * H), jnp.float32),
        mesh=mesh,
        compiler_params=pltpu.CompilerParams(needs_layout_passes=False),
        scratch_types=[
            pltpu.VMEM((BS, 3), jnp.float32),
            pltpu.VMEM((VPAD,), jnp.float32),
            pltpu.VMEM((NF * 3,), jnp.int32),
            pltpu.VMEM((NVUP,), jnp.float32),
            pltpu.VMEM((NVUP,), jnp.float32),
            pltpu.VMEM((NF,), jnp.float32),
            pltpu.VMEM((NF,), jnp.float32),
            pltpu.VMEM((RCH, H), jnp.int32),
            pltpu.VMEM((RCH, 2 * H), jnp.float32),
        ],
    )
    return f(cams, verts, faces_a, fim)


def kernel(src_cams, src_verts, faces, src_fim, tgt_fim):
    del src_fim
    verts = jnp.pad(src_verts.astype(jnp.float32).reshape(BS, NV * 3),
                    ((0, 0), (0, VPAD - NV * 3)))
    out = _run(src_cams.astype(jnp.float32), verts,
               faces.astype(jnp.int32).reshape(-1), tgt_fim.astype(jnp.int32))
    return out.reshape(BS, H, H, 2)
